# Initial kernel scaffold; baseline (speedup 1.0000x reference)
import jax
import jax.numpy as jnp
from jax.experimental import pallas as pl


def kernel(h, edge_index, coord, edge_attr, e_w1, e_b1, e_w2, e_b2, n_w1, n_b1, n_w2, n_b2, c_w1, c_b1, c_w2):
    raise NotImplementedError("write your pallas kernel here")


# speedup vs baseline: 2.4447x; 2.4447x over previous
import functools

import jax
import jax.numpy as jnp
from jax import lax
from jax.experimental import pallas as pl
from jax.experimental.pallas import tpu as pltpu
from jax.experimental.pallas import tpu_sc as plsc

N = 10000
E = 320000
D = 128

NC = 2
NS = 16
NW = NC * NS
EPW = E // NW
C1 = 80
NCH1 = EPW // C1
C3 = 80
NCH3 = EPW // C3
NP = 10240
RPT = NP // NS
SW = 16
NZB = RPT // C3

_mesh = plsc.VectorSubcoreMesh(core_axis_name="c", subcore_axis_name="s")


def _stage0_body(h_ref, wa_ref, wb_ref, b1_ref, hs_ref, ht_ref):
    h = h_ref[...]
    hs_ref[...] = jnp.dot(h, wa_ref[...], preferred_element_type=jnp.float32) + b1_ref[...]
    ht_ref[...] = jnp.dot(h, wb_ref[...], preferred_element_type=jnp.float32)


def _stage0(h, wa, wb, b1):
    blk = 2000
    return pl.pallas_call(
        _stage0_body,
        grid=(N // blk,),
        in_specs=[
            pl.BlockSpec((blk, D), lambda i: (i, 0)),
            pl.BlockSpec((D, D), lambda i: (0, 0)),
            pl.BlockSpec((D, D), lambda i: (0, 0)),
            pl.BlockSpec((1, D), lambda i: (0, 0)),
        ],
        out_specs=[
            pl.BlockSpec((blk, D), lambda i: (i, 0)),
            pl.BlockSpec((blk, D), lambda i: (i, 0)),
        ],
        out_shape=[
            jax.ShapeDtypeStruct((N, D), jnp.float32),
            jax.ShapeDtypeStruct((N, D), jnp.float32),
        ],
    )(h, wa, wb, b1)


def _stage1_body(hs_hbm, ht_hbm, cx_hbm, cy_hbm, cz_hbm, row_hbm, col_hbm,
                 pre_hbm, dx_hbm, dy_hbm, dz_hbm, rad_hbm,
                 row_v, col_v, g_buf, t_buf, crx, cry, crz, ccx, ccy, ccz,
                 rad_v, sem_a, sem_b, sem_c, sem_d):
    wid = lax.axis_index("s") * NC + lax.axis_index("c")
    base0 = wid * EPW

    def chunk(j, carry):
        base = base0 + j * C1
        sl = pl.ds(base, C1)
        pltpu.sync_copy(row_hbm.at[sl], row_v)
        pltpu.sync_copy(col_hbm.at[sl], col_v)
        cp1 = pltpu.async_copy(hs_hbm.at[row_v], g_buf, sem_a)
        cp2 = pltpu.async_copy(ht_hbm.at[col_v], t_buf, sem_b)
        g_r = [pltpu.async_copy(t.at[row_v], d, sem_c)
               for t, d in ((cx_hbm, crx), (cy_hbm, cry), (cz_hbm, crz))]
        g_c = [pltpu.async_copy(t.at[col_v], d, sem_d)
               for t, d in ((cx_hbm, ccx), (cy_hbm, ccy), (cz_hbm, ccz))]
        for cp in g_r + g_c:
            cp.wait()
        def diffgrp(g, c):
            s = pl.ds(g * 16, 16)
            dx = crx[s] - ccx[s]
            dy = cry[s] - ccy[s]
            dz = crz[s] - ccz[s]
            crx[s] = dx
            cry[s] = dy
            crz[s] = dz
            rad_v[s] = dx * dx + dy * dy + dz * dz
            return c

        lax.fori_loop(0, C1 // 16, diffgrp, 0, unroll=5)
        pltpu.sync_copy(crx, dx_hbm.at[sl])
        pltpu.sync_copy(cry, dy_hbm.at[sl])
        pltpu.sync_copy(crz, dz_hbm.at[sl])
        pltpu.sync_copy(rad_v, rad_hbm.at[sl])
        cp1.wait()
        cp2.wait()

        def addrow(r, c):
            for k in range(D // 16):
                s = pl.ds(k * 16, 16)
                g_buf[r, s] = g_buf[r, s] + t_buf[r, s]
            return c

        lax.fori_loop(0, C1, addrow, 0, unroll=2)
        pltpu.sync_copy(g_buf, pre_hbm.at[sl])
        return carry

    lax.fori_loop(0, NCH1, chunk, 0)


@functools.partial(
    pl.kernel,
    out_type=(
        jax.ShapeDtypeStruct((E, D), jnp.float32),
        jax.ShapeDtypeStruct((E,), jnp.float32),
        jax.ShapeDtypeStruct((E,), jnp.float32),
        jax.ShapeDtypeStruct((E,), jnp.float32),
        jax.ShapeDtypeStruct((E,), jnp.float32),
    ),
    mesh=_mesh,
    scratch_types=[
        pltpu.VMEM((C1,), jnp.int32),
        pltpu.VMEM((C1,), jnp.int32),
        pltpu.VMEM((C1, D), jnp.float32),
        pltpu.VMEM((C1, D), jnp.float32),
        pltpu.VMEM((C1,), jnp.float32),
        pltpu.VMEM((C1,), jnp.float32),
        pltpu.VMEM((C1,), jnp.float32),
        pltpu.VMEM((C1,), jnp.float32),
        pltpu.VMEM((C1,), jnp.float32),
        pltpu.VMEM((C1,), jnp.float32),
        pltpu.VMEM((C1,), jnp.float32),
        pltpu.SemaphoreType.DMA,
        pltpu.SemaphoreType.DMA,
        pltpu.SemaphoreType.DMA,
        pltpu.SemaphoreType.DMA,
    ],
)
def _stage1(hs_hbm, ht_hbm, cx_hbm, cy_hbm, cz_hbm, row_hbm, col_hbm, *rest):
    _stage1_body(hs_hbm, ht_hbm, cx_hbm, cy_hbm, cz_hbm, row_hbm, col_hbm, *rest)


def _stage2_body(pre_ref, ea_ref, dx_ref, dy_ref, dz_ref, rad_ref, wk_ref,
                 wr_ref, w2_ref, b2_ref, cw1_ref, cb1_ref, cw2_ref,
                 ef_ref, sm_ref):
    pre = pre_ref[...] + rad_ref[...] * wr_ref[...]
    pre = pre + jnp.dot(ea_ref[...], wk_ref[...], preferred_element_type=jnp.float32)
    t1 = jnp.maximum(pre, 0.0)
    ef = jnp.maximum(jnp.dot(t1, w2_ref[...], preferred_element_type=jnp.float32) + b2_ref[...], 0.0)
    q = jnp.maximum(jnp.dot(ef, cw1_ref[...], preferred_element_type=jnp.float32) + cb1_ref[...], 0.0)
    cm = jnp.dot(q, cw2_ref[...], preferred_element_type=jnp.float32)
    ef_ref[...] = ef
    tx = jnp.clip(cm * dx_ref[...], -100.0, 100.0)
    ty = jnp.clip(cm * dy_ref[...], -100.0, 100.0)
    tz = jnp.clip(cm * dz_ref[...], -100.0, 100.0)
    lane = lax.broadcasted_iota(jnp.int32, (tx.shape[0], SW), 1)
    one = jnp.float32(1.0)
    zero = jnp.float32(0.0)
    sm_ref[...] = jnp.where(
        lane == 0, tx,
        jnp.where(lane == 1, ty,
                  jnp.where(lane == 2, tz,
                            jnp.where(lane == 3, one, zero))))


def _stage2(pre, ea, dxe, dye, dze, rad, wk, wr, w2, b2, cw1, cb1, cw2):
    blk = 2560
    full = lambda i: (0, 0)
    return pl.pallas_call(
        _stage2_body,
        grid=(E // blk,),
        in_specs=[
            pl.BlockSpec((blk, D), lambda i: (i, 0)),
            pl.BlockSpec((blk, D), lambda i: (i, 0)),
            pl.BlockSpec((blk, 1), lambda i: (i, 0)),
            pl.BlockSpec((blk, 1), lambda i: (i, 0)),
            pl.BlockSpec((blk, 1), lambda i: (i, 0)),
            pl.BlockSpec((blk, 1), lambda i: (i, 0)),
            pl.BlockSpec((D, D), full),
            pl.BlockSpec((1, D), full),
            pl.BlockSpec((D, D), full),
            pl.BlockSpec((1, D), full),
            pl.BlockSpec((D, D), full),
            pl.BlockSpec((1, D), full),
            pl.BlockSpec((D, 1), full),
        ],
        out_specs=[
            pl.BlockSpec((blk, D), lambda i: (i, 0)),
            pl.BlockSpec((blk, SW), lambda i: (i, 0)),
        ],
        out_shape=[
            jax.ShapeDtypeStruct((E, D), jnp.float32),
            jax.ShapeDtypeStruct((E, SW), jnp.float32),
        ],
    )(pre, ea, dxe, dye, dze, rad, wk, wr, w2, b2, cw1, cb1, cw2)


def _scatter_body(val_hbm, row_hbm, out_hbm, row_v, buf, nar, acc, expand):
    cid = lax.axis_index("c")
    sid = lax.axis_index("s")
    wid = sid * NC + cid
    base0 = wid * EPW
    rbase = sid * RPT

    z16 = jnp.zeros((16,), jnp.float32)

    def zrow(r, c):
        for k in range(D // 16):
            buf[r, pl.ds(k * 16, 16)] = z16
        return c

    lax.fori_loop(0, C3, zrow, 0, unroll=4)
    for q in range(NZB):
        pltpu.sync_copy(buf, acc.at[pl.ds(rbase + q * C3, C3)])
    plsc.subcore_barrier()

    def chunk(j, carry):
        base = base0 + j * C3
        sl = pl.ds(base, C3)
        pltpu.sync_copy(row_hbm.at[sl], row_v)
        if expand:
            pltpu.sync_copy(val_hbm.at[sl], nar)

            def xrow(r, c):
                buf[r, pl.ds(0, SW)] = nar[r, :]
                return c

            lax.fori_loop(0, C3, xrow, 0, unroll=4)
        else:
            pltpu.sync_copy(val_hbm.at[sl], buf)
        pltpu.sync_copy(buf, acc.at[row_v], add=True)
        return carry

    lax.fori_loop(0, NCH3, chunk, 0)
    plsc.subcore_barrier()
    for q in range(NZB):
        qs = pl.ds(rbase + q * C3, C3)
        hs_ = pl.ds(cid * NP + rbase + q * C3, C3)
        pltpu.sync_copy(acc.at[qs], buf)
        pltpu.sync_copy(buf, out_hbm.at[hs_])


@functools.partial(
    pl.kernel,
    out_type=jax.ShapeDtypeStruct((NC * NP, D), jnp.float32),
    mesh=_mesh,
    scratch_types=[
        pltpu.VMEM((C3,), jnp.int32),
        pltpu.VMEM((C3, D), jnp.float32),
        pltpu.VMEM((C3, SW), jnp.float32),
        pltpu.VMEM_SHARED((NP, D), jnp.float32),
    ],
)
def _stage3e(ef_hbm, row_hbm, pe_hbm, row_v, buf, nar, acc):
    _scatter_body(ef_hbm, row_hbm, pe_hbm, row_v, buf, nar, acc, False)


@functools.partial(
    pl.kernel,
    out_type=jax.ShapeDtypeStruct((NC * NP, D), jnp.float32),
    mesh=_mesh,
    scratch_types=[
        pltpu.VMEM((C3,), jnp.int32),
        pltpu.VMEM((C3, D), jnp.float32),
        pltpu.VMEM((C3, SW), jnp.float32),
        pltpu.VMEM_SHARED((NP, D), jnp.float32),
    ],
)
def _stage3s(sm_hbm, row_hbm, ps_hbm, row_v, buf, nar, acc):
    _scatter_body(sm_hbm, row_hbm, ps_hbm, row_v, buf, nar, acc, True)


def _stage4_body(h_ref, pe_ref, ps_ref, co_ref, n1a_ref, n1b_ref, nb1_ref,
                 n2_ref, nb2_ref, hout_ref, cout_ref):
    h = h_ref[...]
    agg_e = pe_ref[0] + pe_ref[1]
    s = ps_ref[0] + ps_ref[1]
    cnt = jnp.maximum(s[:, 3:4], 1.0)
    aggc = s[:, 0:3] / cnt
    cout_ref[...] = co_ref[...] + aggc
    t = jnp.maximum(
        jnp.dot(h, n1a_ref[...], preferred_element_type=jnp.float32)
        + jnp.dot(agg_e, n1b_ref[...], preferred_element_type=jnp.float32)
        + nb1_ref[...], 0.0)
    hout_ref[...] = h + jnp.dot(t, n2_ref[...], preferred_element_type=jnp.float32) + nb2_ref[...]


def _stage4(h, pe, ps, coord, n1a, n1b, nb1, n2, nb2):
    blk = 2000
    full = lambda i: (0, 0)
    return pl.pallas_call(
        _stage4_body,
        grid=(N // blk,),
        in_specs=[
            pl.BlockSpec((blk, D), lambda i: (i, 0)),
            pl.BlockSpec((NC, blk, D), lambda i: (0, i, 0)),
            pl.BlockSpec((NC, blk, D), lambda i: (0, i, 0)),
            pl.BlockSpec((blk, 3), lambda i: (i, 0)),
            pl.BlockSpec((D, D), full),
            pl.BlockSpec((D, D), full),
            pl.BlockSpec((1, D), full),
            pl.BlockSpec((D, D), full),
            pl.BlockSpec((1, D), full),
        ],
        out_specs=[
            pl.BlockSpec((blk, D), lambda i: (i, 0)),
            pl.BlockSpec((blk, 3), lambda i: (i, 0)),
        ],
        out_shape=[
            jax.ShapeDtypeStruct((N, D), jnp.float32),
            jax.ShapeDtypeStruct((N, 3), jnp.float32),
        ],
    )(h, pe, ps, coord, n1a, n1b, nb1, n2, nb2)


def kernel(h, edge_index, coord, edge_attr, e_w1, e_b1, e_w2, e_b2,
           n_w1, n_b1, n_w2, n_b2, c_w1, c_b1, c_w2):
    f32 = jnp.float32
    wa = e_w1[0:D]
    wb = e_w1[D:2 * D]
    wr = e_w1[2 * D:2 * D + 1]
    wk = e_w1[2 * D + 1:]
    row = edge_index[0]
    col = edge_index[1]
    cx = coord[:, 0]
    cy = coord[:, 1]
    cz = coord[:, 2]

    hs, ht = _stage0(h, wa, wb, e_b1.reshape(1, D))
    pre, dxe, dye, dze, rad = _stage1(hs, ht, cx, cy, cz, row, col)
    ef, sm = _stage2(pre, edge_attr, dxe.reshape(E, 1), dye.reshape(E, 1),
                     dze.reshape(E, 1), rad.reshape(E, 1), wk, wr, e_w2,
                     e_b2.reshape(1, D), c_w1, c_b1.reshape(1, D), c_w2)
    pe = _stage3e(ef, row)
    ps = _stage3s(sm, row)
    h_out, coord_out = _stage4(h, pe.reshape(NC, NP, D), ps.reshape(NC, NP, D),
                               coord, n_w1[0:D], n_w1[D:],
                               n_b1.reshape(1, D), n_w2, n_b2.reshape(1, D))
    return (h_out, coord_out.reshape(N, 3, 1), edge_attr)


# speedup vs baseline: 2.7092x; 1.1082x over previous
import functools

import jax
import jax.numpy as jnp
from jax import lax
from jax.experimental import pallas as pl
from jax.experimental.pallas import tpu as pltpu
from jax.experimental.pallas import tpu_sc as plsc

N = 10000
E = 320000
D = 128

NC = 2
NS = 16
NW = NC * NS
EPW = E // NW
C1 = 80
NCH1 = EPW // C1
C3 = 80
NCH3 = EPW // C3
NP = 10240
RPT = NP // NS
SW = 16
NZB = RPT // C3

_mesh = plsc.VectorSubcoreMesh(core_axis_name="c", subcore_axis_name="s")


def _stage0_body(h_ref, wa_ref, wb_ref, b1_ref, hs_ref, ht_ref):
    h = h_ref[...]
    hs_ref[...] = jnp.dot(h, wa_ref[...], preferred_element_type=jnp.float32) + b1_ref[...]
    ht_ref[...] = jnp.dot(h, wb_ref[...], preferred_element_type=jnp.float32)


def _stage0(h, wa, wb, b1):
    blk = 2000
    return pl.pallas_call(
        _stage0_body,
        grid=(N // blk,),
        in_specs=[
            pl.BlockSpec((blk, D), lambda i: (i, 0)),
            pl.BlockSpec((D, D), lambda i: (0, 0)),
            pl.BlockSpec((D, D), lambda i: (0, 0)),
            pl.BlockSpec((1, D), lambda i: (0, 0)),
        ],
        out_specs=[
            pl.BlockSpec((blk, D), lambda i: (i, 0)),
            pl.BlockSpec((blk, D), lambda i: (i, 0)),
        ],
        out_shape=[
            jax.ShapeDtypeStruct((N, D), jnp.float32),
            jax.ShapeDtypeStruct((N, D), jnp.float32),
        ],
    )(h, wa, wb, b1)


def _stage1_body(hs_hbm, ht_hbm, cx_hbm, cy_hbm, cz_hbm, row_hbm, col_hbm,
                 pre_hbm, prf_hbm, dx_hbm, dy_hbm, dz_hbm, rad_hbm,
                 row_v, col_v, g_buf, t_buf, crx, cry, crz, ccx, ccy, ccz,
                 rad_v, sem_a, sem_b, sem_c, sem_d):
    wid = lax.axis_index("s") * NC + lax.axis_index("c")
    base0 = wid * EPW

    def chunk(j, carry):
        base = base0 + j * C1
        sl = pl.ds(base, C1)
        pltpu.sync_copy(row_hbm.at[sl], row_v)
        pltpu.sync_copy(col_hbm.at[sl], col_v)
        cp1 = pltpu.async_copy(hs_hbm.at[row_v], g_buf, sem_a)
        cp2 = pltpu.async_copy(ht_hbm.at[col_v], t_buf, sem_b)
        g_r = [pltpu.async_copy(t.at[row_v], d, sem_c)
               for t, d in ((cx_hbm, crx), (cy_hbm, cry), (cz_hbm, crz))]
        g_c = [pltpu.async_copy(t.at[col_v], d, sem_d)
               for t, d in ((cx_hbm, ccx), (cy_hbm, ccy), (cz_hbm, ccz))]
        for cp in g_r + g_c:
            cp.wait()
        def diffgrp(g, c):
            s = pl.ds(g * 16, 16)
            dx = crx[s] - ccx[s]
            dy = cry[s] - ccy[s]
            dz = crz[s] - ccz[s]
            crx[s] = dx
            cry[s] = dy
            crz[s] = dz
            rad_v[s] = dx * dx + dy * dy + dz * dz
            return c

        lax.fori_loop(0, C1 // 16, diffgrp, 0, unroll=5)
        pltpu.sync_copy(crx, dx_hbm.at[sl])
        pltpu.sync_copy(cry, dy_hbm.at[sl])
        pltpu.sync_copy(crz, dz_hbm.at[sl])
        pltpu.sync_copy(rad_v, rad_hbm.at[sl])
        cp1.wait()
        cp2.wait()
        pltpu.sync_copy(g_buf, pre_hbm.at[sl])
        pltpu.sync_copy(t_buf, prf_hbm.at[sl])
        return carry

    lax.fori_loop(0, NCH1, chunk, 0)


@functools.partial(
    pl.kernel,
    out_type=(
        jax.ShapeDtypeStruct((E, D), jnp.float32),
        jax.ShapeDtypeStruct((E, D), jnp.float32),
        jax.ShapeDtypeStruct((E,), jnp.float32),
        jax.ShapeDtypeStruct((E,), jnp.float32),
        jax.ShapeDtypeStruct((E,), jnp.float32),
        jax.ShapeDtypeStruct((E,), jnp.float32),
    ),
    mesh=_mesh,
    scratch_types=[
        pltpu.VMEM((C1,), jnp.int32),
        pltpu.VMEM((C1,), jnp.int32),
        pltpu.VMEM((C1, D), jnp.float32),
        pltpu.VMEM((C1, D), jnp.float32),
        pltpu.VMEM((C1,), jnp.float32),
        pltpu.VMEM((C1,), jnp.float32),
        pltpu.VMEM((C1,), jnp.float32),
        pltpu.VMEM((C1,), jnp.float32),
        pltpu.VMEM((C1,), jnp.float32),
        pltpu.VMEM((C1,), jnp.float32),
        pltpu.VMEM((C1,), jnp.float32),
        pltpu.SemaphoreType.DMA,
        pltpu.SemaphoreType.DMA,
        pltpu.SemaphoreType.DMA,
        pltpu.SemaphoreType.DMA,
    ],
)
def _stage1(hs_hbm, ht_hbm, cx_hbm, cy_hbm, cz_hbm, row_hbm, col_hbm, *rest):
    _stage1_body(hs_hbm, ht_hbm, cx_hbm, cy_hbm, cz_hbm, row_hbm, col_hbm, *rest)


def _stage2_body(pre_ref, prf_ref, ea_ref, dx_ref, dy_ref, dz_ref, rad_ref,
                 wk_ref, wr_ref, w2_ref, b2_ref, cw1_ref, cb1_ref, cw2_ref,
                 ef_ref, sm_ref):
    pre = pre_ref[...] + prf_ref[...] + rad_ref[...] * wr_ref[...]
    pre = pre + jnp.dot(ea_ref[...], wk_ref[...], preferred_element_type=jnp.float32)
    t1 = jnp.maximum(pre, 0.0)
    ef = jnp.maximum(jnp.dot(t1, w2_ref[...], preferred_element_type=jnp.float32) + b2_ref[...], 0.0)
    q = jnp.maximum(jnp.dot(ef, cw1_ref[...], preferred_element_type=jnp.float32) + cb1_ref[...], 0.0)
    cm = jnp.dot(q, cw2_ref[...], preferred_element_type=jnp.float32)
    ef_ref[...] = ef
    tx = jnp.clip(cm * dx_ref[...], -100.0, 100.0)
    ty = jnp.clip(cm * dy_ref[...], -100.0, 100.0)
    tz = jnp.clip(cm * dz_ref[...], -100.0, 100.0)
    lane = lax.broadcasted_iota(jnp.int32, (tx.shape[0], SW), 1)
    one = jnp.float32(1.0)
    zero = jnp.float32(0.0)
    sm_ref[...] = jnp.where(
        lane == 0, tx,
        jnp.where(lane == 1, ty,
                  jnp.where(lane == 2, tz,
                            jnp.where(lane == 3, one, zero))))


def _stage2(pre, prf, ea, dxe, dye, dze, rad, wk, wr, w2, b2, cw1, cb1, cw2):
    blk = 2560
    full = lambda i: (0, 0)
    return pl.pallas_call(
        _stage2_body,
        grid=(E // blk,),
        in_specs=[
            pl.BlockSpec((blk, D), lambda i: (i, 0)),
            pl.BlockSpec((blk, D), lambda i: (i, 0)),
            pl.BlockSpec((blk, D), lambda i: (i, 0)),
            pl.BlockSpec((blk, 1), lambda i: (i, 0)),
            pl.BlockSpec((blk, 1), lambda i: (i, 0)),
            pl.BlockSpec((blk, 1), lambda i: (i, 0)),
            pl.BlockSpec((blk, 1), lambda i: (i, 0)),
            pl.BlockSpec((D, D), full),
            pl.BlockSpec((1, D), full),
            pl.BlockSpec((D, D), full),
            pl.BlockSpec((1, D), full),
            pl.BlockSpec((D, D), full),
            pl.BlockSpec((1, D), full),
            pl.BlockSpec((D, 1), full),
        ],
        out_specs=[
            pl.BlockSpec((blk, D), lambda i: (i, 0)),
            pl.BlockSpec((blk, SW), lambda i: (i, 0)),
        ],
        out_shape=[
            jax.ShapeDtypeStruct((E, D), jnp.float32),
            jax.ShapeDtypeStruct((E, SW), jnp.float32),
        ],
    )(pre, prf, ea, dxe, dye, dze, rad, wk, wr, w2, b2, cw1, cb1, cw2)


def _scatter_body(val_hbm, row_hbm, out_hbm, row_v, buf, nar, acc, expand):
    cid = lax.axis_index("c")
    sid = lax.axis_index("s")
    wid = sid * NC + cid
    base0 = wid * EPW
    rbase = sid * RPT

    z16 = jnp.zeros((16,), jnp.float32)

    def zrow(r, c):
        for k in range(D // 16):
            buf[r, pl.ds(k * 16, 16)] = z16
        return c

    lax.fori_loop(0, C3, zrow, 0, unroll=4)
    for q in range(NZB):
        pltpu.sync_copy(buf, acc.at[pl.ds(rbase + q * C3, C3)])
    plsc.subcore_barrier()

    def chunk(j, carry):
        base = base0 + j * C3
        sl = pl.ds(base, C3)
        pltpu.sync_copy(row_hbm.at[sl], row_v)
        if expand:
            pltpu.sync_copy(val_hbm.at[sl], nar)

            def xrow(r, c):
                buf[r, pl.ds(0, SW)] = nar[r, :]
                return c

            lax.fori_loop(0, C3, xrow, 0, unroll=4)
        else:
            pltpu.sync_copy(val_hbm.at[sl], buf)
        pltpu.sync_copy(buf, acc.at[row_v], add=True)
        return carry

    lax.fori_loop(0, NCH3, chunk, 0)
    plsc.subcore_barrier()
    for q in range(NZB):
        qs = pl.ds(rbase + q * C3, C3)
        hs_ = pl.ds(cid * NP + rbase + q * C3, C3)
        pltpu.sync_copy(acc.at[qs], buf)
        pltpu.sync_copy(buf, out_hbm.at[hs_])


@functools.partial(
    pl.kernel,
    out_type=jax.ShapeDtypeStruct((NC * NP, D), jnp.float32),
    mesh=_mesh,
    scratch_types=[
        pltpu.VMEM((C3,), jnp.int32),
        pltpu.VMEM((C3, D), jnp.float32),
        pltpu.VMEM((C3, SW), jnp.float32),
        pltpu.VMEM_SHARED((NP, D), jnp.float32),
    ],
)
def _stage3e(ef_hbm, row_hbm, pe_hbm, row_v, buf, nar, acc):
    _scatter_body(ef_hbm, row_hbm, pe_hbm, row_v, buf, nar, acc, False)


@functools.partial(
    pl.kernel,
    out_type=jax.ShapeDtypeStruct((NC * NP, D), jnp.float32),
    mesh=_mesh,
    scratch_types=[
        pltpu.VMEM((C3,), jnp.int32),
        pltpu.VMEM((C3, D), jnp.float32),
        pltpu.VMEM((C3, SW), jnp.float32),
        pltpu.VMEM_SHARED((NP, D), jnp.float32),
    ],
)
def _stage3s(sm_hbm, row_hbm, ps_hbm, row_v, buf, nar, acc):
    _scatter_body(sm_hbm, row_hbm, ps_hbm, row_v, buf, nar, acc, True)


def _stage4_body(h_ref, pe_ref, ps_ref, co_ref, n1a_ref, n1b_ref, nb1_ref,
                 n2_ref, nb2_ref, hout_ref, cout_ref):
    h = h_ref[...]
    agg_e = pe_ref[0] + pe_ref[1]
    s = ps_ref[0] + ps_ref[1]
    cnt = jnp.maximum(s[:, 3:4], 1.0)
    aggc = s[:, 0:3] / cnt
    cout_ref[...] = co_ref[...] + aggc
    t = jnp.maximum(
        jnp.dot(h, n1a_ref[...], preferred_element_type=jnp.float32)
        + jnp.dot(agg_e, n1b_ref[...], preferred_element_type=jnp.float32)
        + nb1_ref[...], 0.0)
    hout_ref[...] = h + jnp.dot(t, n2_ref[...], preferred_element_type=jnp.float32) + nb2_ref[...]


def _stage4(h, pe, ps, coord, n1a, n1b, nb1, n2, nb2):
    blk = 2000
    full = lambda i: (0, 0)
    return pl.pallas_call(
        _stage4_body,
        grid=(N // blk,),
        in_specs=[
            pl.BlockSpec((blk, D), lambda i: (i, 0)),
            pl.BlockSpec((NC, blk, D), lambda i: (0, i, 0)),
            pl.BlockSpec((NC, blk, D), lambda i: (0, i, 0)),
            pl.BlockSpec((blk, 3), lambda i: (i, 0)),
            pl.BlockSpec((D, D), full),
            pl.BlockSpec((D, D), full),
            pl.BlockSpec((1, D), full),
            pl.BlockSpec((D, D), full),
            pl.BlockSpec((1, D), full),
        ],
        out_specs=[
            pl.BlockSpec((blk, D), lambda i: (i, 0)),
            pl.BlockSpec((blk, 3), lambda i: (i, 0)),
        ],
        out_shape=[
            jax.ShapeDtypeStruct((N, D), jnp.float32),
            jax.ShapeDtypeStruct((N, 3), jnp.float32),
        ],
    )(h, pe, ps, coord, n1a, n1b, nb1, n2, nb2)


def kernel(h, edge_index, coord, edge_attr, e_w1, e_b1, e_w2, e_b2,
           n_w1, n_b1, n_w2, n_b2, c_w1, c_b1, c_w2):
    f32 = jnp.float32
    wa = e_w1[0:D]
    wb = e_w1[D:2 * D]
    wr = e_w1[2 * D:2 * D + 1]
    wk = e_w1[2 * D + 1:]
    row = edge_index[0]
    col = edge_index[1]
    cx = coord[:, 0]
    cy = coord[:, 1]
    cz = coord[:, 2]

    hs, ht = _stage0(h, wa, wb, e_b1.reshape(1, D))
    pre, prf, dxe, dye, dze, rad = _stage1(hs, ht, cx, cy, cz, row, col)
    ef, sm = _stage2(pre, prf, edge_attr, dxe.reshape(E, 1), dye.reshape(E, 1),
                     dze.reshape(E, 1), rad.reshape(E, 1), wk, wr, e_w2,
                     e_b2.reshape(1, D), c_w1, c_b1.reshape(1, D), c_w2)
    pe = _stage3e(ef, row)
    ps = _stage3s(sm, row)
    h_out, coord_out = _stage4(h, pe.reshape(NC, NP, D), ps.reshape(NC, NP, D),
                               coord, n_w1[0:D], n_w1[D:],
                               n_b1.reshape(1, D), n_w2, n_b2.reshape(1, D))
    return (h_out, coord_out.reshape(N, 3, 1), edge_attr)
